# Initial kernel scaffold; baseline (speedup 1.0000x reference)
import jax
import jax.numpy as jnp
from jax.experimental import pallas as pl


def kernel(queries, keys, values):
    raise NotImplementedError("write your pallas kernel here")


# speedup vs baseline: 1.5440x; 1.5440x over previous
import functools

import jax
import jax.numpy as jnp
from jax import lax
from jax.experimental import pallas as pl
from jax.experimental.pallas import tpu as pltpu
from jax.experimental.pallas import tpu_sc as plsc

D_MODEL = 64
TOP_K = 5
KPAD = 8
K_TILE = 2048

NUM_CORES = 2
NUM_SUBCORES = 16
NUM_WORKERS = NUM_CORES * NUM_SUBCORES
LANES = 16

_NEG_INF = float("-inf")
_I32_MAX = jnp.iinfo(jnp.int32).max


def _top5_of(vals, idxs):
    out_v, out_i = [], []
    for t in range(TOP_K):
        m = jnp.max(vals, axis=1, keepdims=True)
        hit = vals == m
        im = jnp.min(jnp.where(hit, idxs, _I32_MAX), axis=1, keepdims=True)
        out_v.append(m)
        out_i.append(im)
        if t < TOP_K - 1:
            vals = jnp.where(idxs == im, _NEG_INF, vals)
    return jnp.concatenate(out_v, axis=1), jnp.concatenate(out_i, axis=1)


def _topk_body(q_ref, k_ref, w_ref, oidx_ref, opar_ref, vals_ref, idx_ref, *,
               n_keys, n_tiles, q_rows):
    i = pl.program_id(0)

    @pl.when(i == 0)
    def _init():
        vals_ref[...] = jnp.full((q_rows, KPAD), _NEG_INF, jnp.float32)
        idx_ref[...] = jnp.zeros((q_rows, KPAD), jnp.int32)

    q = q_ref[...]
    k = k_ref[...]
    sim = lax.dot_general(q, k, (((1,), (1,)), ((), ())),
                          precision=lax.Precision.DEFAULT,
                          preferred_element_type=jnp.float32)
    cols = i * K_TILE + lax.broadcasted_iota(jnp.int32, (q_rows, K_TILE), 1)
    sim = jnp.where(cols < n_keys, sim, _NEG_INF)

    bv, bi = _top5_of(sim, cols)

    pad_v = jnp.full((q_rows, KPAD - TOP_K), _NEG_INF, jnp.float32)
    pad_i = jnp.zeros((q_rows, KPAD - TOP_K), jnp.int32)
    cand_v = jnp.concatenate([vals_ref[...], bv, pad_v], axis=1)
    cand_i = jnp.concatenate([idx_ref[...], bi, pad_i], axis=1)
    mv, mi = _top5_of(cand_v, cand_i)

    vals_ref[...] = jnp.concatenate([mv, pad_v], axis=1)
    idx_ref[...] = jnp.concatenate([mi, pad_i], axis=1)

    @pl.when(i == n_tiles - 1)
    def _finish():
        tv = mv * (1.0 / jnp.sqrt(jnp.float32(D_MODEL)))
        m = jnp.max(tv, axis=1, keepdims=True)
        e = jnp.exp(tv - m)
        w = e / jnp.sum(e, axis=1, keepdims=True)
        w_ref[...] = jnp.concatenate(
            [w, jnp.zeros((q_rows, KPAD - TOP_K), jnp.float32)], axis=1)
        full_i = jnp.concatenate([mi, pad_i], axis=1)
        oidx_ref[...] = lax.shift_right_logical(full_i, 1)
        opar_ref[...] = lax.bitwise_and(full_i, 1)


def _topk_stage(queries, keys, interpret=False):
    q_rows = queries.shape[0]
    n_keys = keys.shape[0]
    n_tiles = (n_keys + K_TILE - 1) // K_TILE
    n_pad = n_tiles * K_TILE - n_keys
    if n_pad:
        keys = jnp.pad(keys, ((0, n_pad), (0, 0)))
    body = functools.partial(_topk_body, n_keys=n_keys, n_tiles=n_tiles,
                             q_rows=q_rows)
    return pl.pallas_call(
        body,
        grid=(n_tiles,),
        in_specs=[
            pl.BlockSpec((q_rows, D_MODEL), lambda i: (0, 0)),
            pl.BlockSpec((K_TILE, D_MODEL), lambda i: (i, 0)),
        ],
        out_specs=[
            pl.BlockSpec((q_rows, KPAD), lambda i: (0, 0)),
            pl.BlockSpec((q_rows, KPAD), lambda i: (0, 0)),
            pl.BlockSpec((q_rows, KPAD), lambda i: (0, 0)),
        ],
        out_shape=[
            jax.ShapeDtypeStruct((q_rows, KPAD), jnp.float32),
            jax.ShapeDtypeStruct((q_rows, KPAD), jnp.int32),
            jax.ShapeDtypeStruct((q_rows, KPAD), jnp.int32),
        ],
        scratch_shapes=[
            pltpu.VMEM((q_rows, KPAD), jnp.float32),
            pltpu.VMEM((q_rows, KPAD), jnp.int32),
        ],
        compiler_params=pltpu.CompilerParams(
            dimension_semantics=("arbitrary",)),
        interpret=interpret,
    )(queries, keys)


def _blend_body(q_hbm, w_hbm, idx_hbm, par_hbm, values_hbm, out_hbm,
                idx_a, idx_b, w_v, par_v, rows_a, rows_b, q_v, o_v,
                sem_a, sem_b, *, bpw):
    wid = lax.axis_index("s") * NUM_CORES + lax.axis_index("c")
    base = wid * bpw
    half = bpw * KPAD // 2
    wrows = bpw * KPAD // LANES
    pltpu.sync_copy(idx_hbm.at[pl.ds(base * KPAD, half)], idx_a)
    pltpu.sync_copy(idx_hbm.at[pl.ds(base * KPAD + half, half)], idx_b)
    pltpu.sync_copy(w_hbm.at[pl.ds(wid * wrows, wrows)], w_v)
    pltpu.sync_copy(par_hbm.at[pl.ds(wid * wrows, wrows)], par_v)
    pltpu.sync_copy(q_hbm.at[pl.ds(base, bpw)], q_v)
    cp_a = pltpu.async_copy(values_hbm.at[idx_a], rows_a, sem_a)
    cp_b = pltpu.async_copy(values_hbm.at[idx_b], rows_b, sem_b)
    cp_a.wait()
    cp_b.wait()

    qp = LANES // KPAD

    def make_pbody(rows_ref, p0):
        def pbody(p, carry):
            wvec = w_v[p, :]
            pvec = par_v[p, :]
            for sub in range(qp):
                qi = p * qp + sub
                for c in range(D_MODEL // LANES):
                    acc = q_v[qi, pl.ds(c * LANES, LANES)]
                    for kk in range(KPAD):
                        wgt = wvec[sub * KPAD + kk]
                        par = pvec[sub * KPAD + kk]
                        slot = (p - p0) * LANES + sub * KPAD + kk
                        lo = rows_ref[slot, pl.ds(c * LANES, LANES)]
                        hi = rows_ref[slot, pl.ds(D_MODEL + c * LANES, LANES)]
                        acc = acc + wgt * jnp.where(par == 1, hi, lo)
                    o_v[qi, pl.ds(c * LANES, LANES)] = acc
            return carry
        return pbody

    lax.fori_loop(0, wrows // 2, make_pbody(rows_a, 0), 0)
    lax.fori_loop(wrows // 2, wrows, make_pbody(rows_b, wrows // 2), 0)
    pltpu.sync_copy(o_v, out_hbm.at[pl.ds(base, bpw)])


def _blend_stage(queries, w_2d, idx_flat, par_2d, values_pairs,
                 interpret=False):
    q_rows = queries.shape[0]
    bpw = q_rows // NUM_WORKERS
    half = bpw * KPAD // 2
    wrows = bpw * KPAD // LANES
    mesh = plsc.VectorSubcoreMesh(core_axis_name="c", subcore_axis_name="s")
    body = functools.partial(_blend_body, bpw=bpw)
    return pl.kernel(
        body,
        out_type=jax.ShapeDtypeStruct((q_rows, D_MODEL), jnp.float32),
        mesh=mesh,
        scratch_types=[
            pltpu.VMEM((half,), jnp.int32),
            pltpu.VMEM((half,), jnp.int32),
            pltpu.VMEM((wrows, LANES), jnp.float32),
            pltpu.VMEM((wrows, LANES), jnp.int32),
            pltpu.VMEM((half, 2 * D_MODEL), jnp.float32),
            pltpu.VMEM((half, 2 * D_MODEL), jnp.float32),
            pltpu.VMEM((bpw, D_MODEL), jnp.float32),
            pltpu.VMEM((bpw, D_MODEL), jnp.float32),
            pltpu.SemaphoreType.DMA,
            pltpu.SemaphoreType.DMA,
        ],
        interpret=interpret,
    )(queries, w_2d, idx_flat, par_2d, values_pairs)


def kernel(queries, keys, values):
    w, oidx, opar = _topk_stage(queries, keys)
    w_2d = w.reshape(-1, LANES)
    idx_flat = oidx.reshape(-1)
    par_2d = opar.reshape(-1, LANES)
    values_pairs = values.reshape(-1, 2 * D_MODEL)
    return _blend_stage(queries, w_2d, idx_flat, par_2d, values_pairs)


# speedup vs baseline: 3.1343x; 2.0299x over previous
import functools

import jax
import jax.numpy as jnp
from jax import lax
from jax.experimental import pallas as pl
from jax.experimental.pallas import tpu as pltpu
from jax.experimental.pallas import tpu_sc as plsc

D_MODEL = 64
TOP_K = 5
KPAD = 8
K_TILE = 2048

NUM_CORES = 2
NUM_SUBCORES = 16
NUM_WORKERS = NUM_CORES * NUM_SUBCORES
LANES = 16

_NEG_INF = float("-inf")
_I32_MAX = jnp.iinfo(jnp.int32).max


def _top5_of(vals, idxs):
    out_v, out_i = [], []
    for t in range(TOP_K):
        m = jnp.max(vals, axis=1, keepdims=True)
        hit = vals == m
        im = jnp.min(jnp.where(hit, idxs, _I32_MAX), axis=1, keepdims=True)
        out_v.append(m)
        out_i.append(im)
        if t < TOP_K - 1:
            vals = jnp.where(idxs == im, _NEG_INF, vals)
    return jnp.concatenate(out_v, axis=1), jnp.concatenate(out_i, axis=1)


GROUP = 16
N_PAIRS = TOP_K * GROUP // 2


def _gmax_body(q_ref, k_ref, gm_ref, *, n_keys, q_rows):
    i = pl.program_id(0)
    q = q_ref[...]
    k = k_ref[...]
    sim = lax.dot_general(k, q, (((1,), (1,)), ((), ())),
                          precision=lax.Precision.DEFAULT,
                          preferred_element_type=jnp.float32)
    rows = i * K_TILE + lax.broadcasted_iota(jnp.int32, (K_TILE, q_rows), 0)
    sim = jnp.where(rows < n_keys, sim, _NEG_INF)
    gm_ref[...] = jnp.max(
        sim.reshape(K_TILE // GROUP, GROUP, q_rows), axis=1)


def _gmax_stage(queries, keys_padded, n_keys, interpret=False):
    q_rows = queries.shape[0]
    n_tiles = keys_padded.shape[0] // K_TILE
    body = functools.partial(_gmax_body, n_keys=n_keys, q_rows=q_rows)
    return pl.pallas_call(
        body,
        grid=(n_tiles,),
        in_specs=[
            pl.BlockSpec((q_rows, D_MODEL), lambda i: (0, 0)),
            pl.BlockSpec((K_TILE, D_MODEL), lambda i: (i, 0)),
        ],
        out_specs=pl.BlockSpec((K_TILE // GROUP, q_rows), lambda i: (i, 0)),
        out_shape=jax.ShapeDtypeStruct(
            (n_tiles * K_TILE // GROUP, q_rows), jnp.float32),
        compiler_params=pltpu.CompilerParams(
            dimension_semantics=("arbitrary",)),
        interpret=interpret,
    )(queries, keys_padded)


def _groupsel_body(gm_ref, cand_ref, *, q_rows, n_groups):
    gm = gm_ref[...]
    gidx = lax.broadcasted_iota(jnp.int32, (q_rows, n_groups), 1)
    _, gi = _top5_of(gm, gidx)
    piota = lax.broadcasted_iota(jnp.int32, (q_rows, GROUP // 2), 1)
    chunks = [gi[:, j:j + 1] * (GROUP // 2) + piota for j in range(TOP_K)]
    cand_ref[...] = jnp.concatenate(chunks, axis=1)


def _groupsel_stage(gm, interpret=False):
    q_rows, n_groups = gm.shape
    body = functools.partial(_groupsel_body, q_rows=q_rows,
                             n_groups=n_groups)
    return pl.pallas_call(
        body,
        out_shape=jax.ShapeDtypeStruct((q_rows, N_PAIRS), jnp.int32),
        interpret=interpret,
    )(gm)


def _keygather_body(cand_hbm, keys_hbm, ck_hbm, idx_v, buf_a, buf_b,
                    sem_a, sem_b, *, rows_per_w):
    wid = lax.axis_index("s") * NUM_CORES + lax.axis_index("c")
    base = wid * rows_per_w
    n_chunks = rows_per_w // 128
    pltpu.sync_copy(cand_hbm.at[pl.ds(base, rows_per_w)], idx_v)

    def chunk(c, buf, sem):
        cp = pltpu.async_copy(
            keys_hbm.at[idx_v.at[pl.ds(c * 128, 128)]], buf, sem)
        cp.wait()
        pltpu.sync_copy(buf, ck_hbm.at[pl.ds(base + c * 128, 128)])

    def cbody(c, carry):
        chunk(c, buf_a, sem_a)
        return carry

    lax.fori_loop(0, n_chunks, cbody, 0)


def _keygather_stage(cand_flat, keys_pairs, interpret=False):
    n_rows = cand_flat.shape[0]
    rows_per_w = n_rows // NUM_WORKERS
    mesh = plsc.VectorSubcoreMesh(core_axis_name="c", subcore_axis_name="s")
    body = functools.partial(_keygather_body, rows_per_w=rows_per_w)
    return pl.kernel(
        body,
        out_type=jax.ShapeDtypeStruct((n_rows, 2 * D_MODEL), jnp.float32),
        mesh=mesh,
        scratch_types=[
            pltpu.VMEM((rows_per_w,), jnp.int32),
            pltpu.VMEM((128, 2 * D_MODEL), jnp.float32),
            pltpu.VMEM((128, 2 * D_MODEL), jnp.float32),
            pltpu.SemaphoreType.DMA,
            pltpu.SemaphoreType.DMA,
        ],
        interpret=interpret,
    )(cand_flat, keys_pairs)


def _candsel_body(q_ref, ck_ref, cand_ref, w_ref, oidx_ref, opar_ref, *,
                  qb):
    qv = q_ref[...].astype(jnp.bfloat16).astype(jnp.float32)
    ck = ck_ref[...].astype(jnp.bfloat16).astype(jnp.float32)
    qb3 = qv[:, None, :]
    s0 = jnp.sum(ck[:, :, :D_MODEL] * qb3, axis=2)
    s1 = jnp.sum(ck[:, :, D_MODEL:] * qb3, axis=2)
    cp = cand_ref[...]
    svals = jnp.concatenate([s0, s1], axis=1)
    sidx = jnp.concatenate([cp * 2, cp * 2 + 1], axis=1)
    tv, ti = _top5_of(svals, sidx)
    tv = tv * (1.0 / jnp.sqrt(jnp.float32(D_MODEL)))
    m = jnp.max(tv, axis=1, keepdims=True)
    e = jnp.exp(tv - m)
    w = e / jnp.sum(e, axis=1, keepdims=True)
    pad_f = jnp.zeros((qb, KPAD - TOP_K), jnp.float32)
    pad_i = jnp.zeros((qb, KPAD - TOP_K), jnp.int32)
    w_ref[...] = jnp.concatenate([w, pad_f], axis=1)
    full_i = jnp.concatenate([ti, pad_i], axis=1)
    oidx_ref[...] = lax.shift_right_logical(full_i, 1)
    opar_ref[...] = lax.bitwise_and(full_i, 1)


def _candsel_stage(queries, ck3, cand, interpret=False):
    q_rows = queries.shape[0]
    qb = 256
    grid = (q_rows // qb,)
    body = functools.partial(_candsel_body, qb=qb)
    return pl.pallas_call(
        body,
        grid=grid,
        in_specs=[
            pl.BlockSpec((qb, D_MODEL), lambda i: (i, 0)),
            pl.BlockSpec((qb, N_PAIRS, 2 * D_MODEL), lambda i: (i, 0, 0)),
            pl.BlockSpec((qb, N_PAIRS), lambda i: (i, 0)),
        ],
        out_specs=[
            pl.BlockSpec((qb, KPAD), lambda i: (i, 0)),
            pl.BlockSpec((qb, KPAD), lambda i: (i, 0)),
            pl.BlockSpec((qb, KPAD), lambda i: (i, 0)),
        ],
        out_shape=[
            jax.ShapeDtypeStruct((q_rows, KPAD), jnp.float32),
            jax.ShapeDtypeStruct((q_rows, KPAD), jnp.int32),
            jax.ShapeDtypeStruct((q_rows, KPAD), jnp.int32),
        ],
        interpret=interpret,
    )(queries, ck3, cand)


def _blend_body(q_hbm, w_hbm, idx_hbm, par_hbm, values_hbm, out_hbm,
                idx_a, idx_b, w_v, par_v, rows_a, rows_b, q_v, o_v,
                sem_a, sem_b, *, bpw):
    wid = lax.axis_index("s") * NUM_CORES + lax.axis_index("c")
    base = wid * bpw
    half = bpw * KPAD // 2
    wrows = bpw * KPAD // LANES
    pltpu.sync_copy(idx_hbm.at[pl.ds(base * KPAD, half)], idx_a)
    pltpu.sync_copy(idx_hbm.at[pl.ds(base * KPAD + half, half)], idx_b)
    pltpu.sync_copy(w_hbm.at[pl.ds(wid * wrows, wrows)], w_v)
    pltpu.sync_copy(par_hbm.at[pl.ds(wid * wrows, wrows)], par_v)
    pltpu.sync_copy(q_hbm.at[pl.ds(base, bpw)], q_v)
    cp_a = pltpu.async_copy(values_hbm.at[idx_a], rows_a, sem_a)
    cp_b = pltpu.async_copy(values_hbm.at[idx_b], rows_b, sem_b)
    cp_a.wait()
    cp_b.wait()

    qp = LANES // KPAD

    def make_pbody(rows_ref, p0):
        def pbody(p, carry):
            wvec = w_v[p, :]
            pvec = par_v[p, :]
            for sub in range(qp):
                qi = p * qp + sub
                for c in range(D_MODEL // LANES):
                    acc = q_v[qi, pl.ds(c * LANES, LANES)]
                    for kk in range(KPAD):
                        wgt = wvec[sub * KPAD + kk]
                        par = pvec[sub * KPAD + kk]
                        slot = (p - p0) * LANES + sub * KPAD + kk
                        lo = rows_ref[slot, pl.ds(c * LANES, LANES)]
                        hi = rows_ref[slot, pl.ds(D_MODEL + c * LANES, LANES)]
                        acc = acc + wgt * jnp.where(par == 1, hi, lo)
                    o_v[qi, pl.ds(c * LANES, LANES)] = acc
            return carry
        return pbody

    lax.fori_loop(0, wrows // 2, make_pbody(rows_a, 0), 0)
    lax.fori_loop(wrows // 2, wrows, make_pbody(rows_b, wrows // 2), 0)
    pltpu.sync_copy(o_v, out_hbm.at[pl.ds(base, bpw)])


def _blend_stage(queries, w_2d, idx_flat, par_2d, values_pairs,
                 interpret=False):
    q_rows = queries.shape[0]
    bpw = q_rows // NUM_WORKERS
    half = bpw * KPAD // 2
    wrows = bpw * KPAD // LANES
    mesh = plsc.VectorSubcoreMesh(core_axis_name="c", subcore_axis_name="s")
    body = functools.partial(_blend_body, bpw=bpw)
    return pl.kernel(
        body,
        out_type=jax.ShapeDtypeStruct((q_rows, D_MODEL), jnp.float32),
        mesh=mesh,
        scratch_types=[
            pltpu.VMEM((half,), jnp.int32),
            pltpu.VMEM((half,), jnp.int32),
            pltpu.VMEM((wrows, LANES), jnp.float32),
            pltpu.VMEM((wrows, LANES), jnp.int32),
            pltpu.VMEM((half, 2 * D_MODEL), jnp.float32),
            pltpu.VMEM((half, 2 * D_MODEL), jnp.float32),
            pltpu.VMEM((bpw, D_MODEL), jnp.float32),
            pltpu.VMEM((bpw, D_MODEL), jnp.float32),
            pltpu.SemaphoreType.DMA,
            pltpu.SemaphoreType.DMA,
        ],
        interpret=interpret,
    )(queries, w_2d, idx_flat, par_2d, values_pairs)


def kernel(queries, keys, values):
    n_keys = keys.shape[0]
    n_tiles = (n_keys + K_TILE - 1) // K_TILE
    n_pad = n_tiles * K_TILE - n_keys
    keys_padded = jnp.pad(keys, ((0, n_pad), (0, 0))) if n_pad else keys
    gm_t = _gmax_stage(queries, keys_padded, n_keys)
    cand = _groupsel_stage(gm_t.T)
    keys_pairs = keys_padded.reshape(-1, 2 * D_MODEL)
    ck = _keygather_stage(cand.reshape(-1), keys_pairs)
    ck3 = ck.reshape(queries.shape[0], N_PAIRS, 2 * D_MODEL)
    w, oidx, opar = _candsel_stage(queries, ck3, cand)
    w_2d = w.reshape(-1, LANES)
    idx_flat = oidx.reshape(-1)
    par_2d = opar.reshape(-1, LANES)
    values_pairs = values.reshape(-1, 2 * D_MODEL)
    return _blend_stage(queries, w_2d, idx_flat, par_2d, values_pairs)
